# baseline (speedup 1.0000x reference)
import functools

import jax
import jax.numpy as jnp
from jax import lax
from jax.experimental import pallas as pl
from jax.experimental.pallas import tpu as pltpu
from jax.experimental.pallas import tpu_sc as plsc

N1 = 10000; E1 = 320000; N2 = 1184; E2 = 175232
D = 128; H = 4; NR = 148; B = 8; OUT = 2; HID = 1000
NSEG = B * NR

NC, NS = 2, 16
NW = NC * NS
CK = 128

NP1 = 10112
NP2 = 1280
NJ = NP1 + NP2
NPSEG = 1280
NJSEG = 2 * NPSEG

EJ = E1 + E2
NCH_AGG = 124
EPADJ = NW * CK * NCH_AGG
NRING = 4
RP1 = 12288
RP2 = 4096
NCH_POOL = (RP1 + RP2) // (NW * CK)

HROWS = 128
SCK = 2048
E_SCAT = E1 + E2 + N1 + N2
SCHUNKS = -(-E_SCAT // (NW * SCK))
E_SCAT_PAD = NW * SCK * SCHUNKS


@functools.lru_cache(maxsize=None)
def _get_mesh():
    return plsc.VectorSubcoreMesh(core_axis_name="c", subcore_axis_name="s",
                                  num_cores=NC, num_subcores=NS)


def _worker_id():
    return lax.axis_index("c") * NS + lax.axis_index("s")


def _zero_vmem_rows(ref, nrows):
    zeros = jnp.zeros((16,), jnp.float32)

    def body(r, _):
        for g in range(8):
            ref[r, pl.ds(16 * g, 16)] = zeros
        return _

    lax.fori_loop(0, nrows, body, None)


def _zero_shared_slice(shared, zbuf, zrows, row0, nrows):
    off = 0
    while off < nrows:
        sz = min(zrows, nrows - off)
        pltpu.sync_copy(zbuf.at[pl.ds(0, sz)],
                        shared.at[pl.ds(row0 + off, sz)])
        off += sz


def _scalar_scatter_body(idx_hbm, w_hbm, out_hbm, hist, idxb, wb, rowidx, accum):
    c = lax.axis_index("c")
    s = lax.axis_index("s")
    wid = _worker_id()

    _zero_vmem_rows(hist, HROWS)
    pltpu.sync_copy(hist.at[pl.ds(0, 8)], accum.at[pl.ds(s * 8, 8)])
    for g in range(HROWS // 16):
        rowidx[pl.ds(16 * g, 16)] = lax.iota(jnp.int32, 16) + 16 * g
    plsc.subcore_barrier()

    lanes = lax.iota(jnp.int32, 16)

    def chunk(j, _):
        base = (wid * SCHUNKS + j) * SCK
        pltpu.sync_copy(idx_hbm.at[pl.ds(base, SCK)], idxb)
        pltpu.sync_copy(w_hbm.at[pl.ds(base, SCK)], wb)

        def grp(t, _):
            iv = idxb[pl.ds(16 * t, 16)]
            wv = wb[pl.ds(16 * t, 16)]
            for l in range(16):
                i = iv[l]
                r = lax.shift_right_logical(i, 7)
                col = lax.bitwise_and(i, 127)
                colg = lax.bitwise_and(col, 112)
                vec = jnp.where(lanes == col - colg, wv[l], 0.0)
                plsc.addupdate(hist.at[r, pl.ds(colg, 16)], vec)
            return _

        lax.fori_loop(0, SCK // 16, grp, None)
        return _

    lax.fori_loop(0, SCHUNKS, chunk, None)
    pltpu.sync_copy(hist, accum.at[rowidx], add=True)
    plsc.subcore_barrier()
    pltpu.sync_copy(accum.at[pl.ds(s * 8, 8)], out_hbm.at[c, pl.ds(s * 8, 8)])


@functools.lru_cache(maxsize=None)
def _get_scalar_scatter():
    return pl.kernel(
        _scalar_scatter_body,
        out_type=jax.ShapeDtypeStruct((NC, HROWS, 128), jnp.float32),
        mesh=_get_mesh(),
        scratch_types=[
            pltpu.VMEM((HROWS, 128), jnp.float32),
            pltpu.VMEM((SCK,), jnp.int32),
            pltpu.VMEM((SCK,), jnp.float32),
            pltpu.VMEM((HROWS,), jnp.int32),
            pltpu.VMEM_SHARED((HROWS, 128), jnp.float32),
        ],
    )


@functools.lru_cache(maxsize=None)
def _make_rowagg(nrows, nchunks, gather):
    rps = nrows // NS
    assert nchunks % 2 == 0 and rps % 8 == 0

    def body(*refs):
        if gather:
            y_hbm, e_hbm, w_hbm, out_hbm = refs[:4]
            ia, wring, r0, r1, gs0, gs1, accum = refs[4:11]
            isems = refs[11:11 + NRING]
            wsems = refs[11 + NRING:11 + 2 * NRING]
        else:
            y_hbm, d_hbm, out_hbm = refs[:3]
            dall, r0, r1, gs0, gs1, accum = refs[3:]
        c = lax.axis_index("c")
        s = lax.axis_index("s")
        wid = _worker_id()

        if not gather:
            pltpu.sync_copy(d_hbm.at[wid], dall)
        _zero_vmem_rows(r0, CK)
        _zero_shared_slice(accum, r0, CK, s * rps, rps)
        plsc.subcore_barrier()

        def idx_start(j, b):
            pltpu.async_copy(e_hbm.at[wid, j], ia.at[b], isems[b])
            pltpu.async_copy(w_hbm.at[wid, j], wring.at[b], wsems[b])

        def idx_wait(j, b):
            pltpu.make_async_copy(e_hbm.at[wid, j], ia.at[b], isems[b]).wait()
            pltpu.make_async_copy(w_hbm.at[wid, j], wring.at[b],
                                  wsems[b]).wait()

        def g_start(j, b, rbuf, gsem):
            if gather:
                pltpu.async_copy(y_hbm.at[ia.at[b, 0]], rbuf, gsem)
            else:
                base = (wid * nchunks + j) * CK
                pltpu.async_copy(y_hbm.at[pl.ds(base, CK)], rbuf, gsem)

        def g_wait(j, b, rbuf, gsem):
            if gather:
                pltpu.make_async_copy(y_hbm.at[ia.at[b, 0]], rbuf, gsem).wait()
            else:
                base = (wid * nchunks + j) * CK
                pltpu.make_async_copy(y_hbm.at[pl.ds(base, CK)], rbuf,
                                      gsem).wait()

        def scale_scatter(j, b, rbuf):
            if gather:
                def scale(t, _):
                    wv = wring[b, pl.ds(16 * t, 16)]
                    for l in range(16):
                        wk = wv[l]
                        rr = 16 * t + l
                        for g in range(8):
                            rbuf[rr, pl.ds(16 * g, 16)] = (
                                rbuf[rr, pl.ds(16 * g, 16)] * wk)
                    return _

                lax.fori_loop(0, CK // 16, scale, None)
                pltpu.sync_copy(rbuf, accum.at[ia.at[b, 1]], add=True)
            else:
                pltpu.sync_copy(rbuf, accum.at[dall.at[j]], add=True)

        if gather:
            assert nchunks % NRING == 0
            for b in range(NRING):
                idx_start(b, b)

            def quad(qq, _):
                j0 = NRING * qq
                for u in range(NRING):
                    j = j0 + u
                    idx_wait(j, u)
                    g_start(j, u, r0, gs0)
                    g_wait(j, u, r0, gs0)
                    scale_scatter(j, u, r0)

                    @pl.when(j + NRING < nchunks)
                    def _(j=j, u=u):
                        idx_start(j + NRING, u)

                return _

            lax.fori_loop(0, nchunks // NRING, quad, None)
        else:
            g_start(0, 0, r0, gs0)

            def pair(jj, _):
                j0 = 2 * jj
                j1 = j0 + 1
                g_start(j1, 0, r1, gs1)
                g_wait(j0, 0, r0, gs0)
                scale_scatter(j0, 0, r0)

                @pl.when(j0 + 2 < nchunks)
                def _():
                    g_start(j0 + 2, 0, r0, gs0)

                g_wait(j1, 0, r1, gs1)
                scale_scatter(j1, 0, r1)
                return _

            lax.fori_loop(0, nchunks // 2, pair, None)

        plsc.subcore_barrier()
        off = 0
        while off < rps:
            sz = min(CK, rps - off)
            pltpu.sync_copy(accum.at[pl.ds(s * rps + off, sz)],
                            out_hbm.at[c, pl.ds(s * rps + off, sz)])
            off += sz

    scratch = []
    if gather:
        scratch.append(pltpu.VMEM((NRING, 2, CK), jnp.int32))
        scratch.append(pltpu.VMEM((NRING, CK), jnp.float32))
    else:
        scratch.append(pltpu.VMEM((nchunks, CK), jnp.int32))
    scratch += [
        pltpu.VMEM((CK, 128), jnp.float32),
        pltpu.VMEM((CK, 128), jnp.float32),
        pltpu.SemaphoreType.DMA,
        pltpu.SemaphoreType.DMA,
        pltpu.VMEM_SHARED((nrows, 128), jnp.float32),
    ]
    if gather:
        scratch += [pltpu.SemaphoreType.DMA] * (2 * NRING)
    return pl.kernel(
        body,
        out_type=jax.ShapeDtypeStruct((NC, nrows, 128), jnp.float32),
        mesh=_get_mesh(),
        scratch_types=scratch,
    )


def _dinv_of(degp):
    deg = degp[0] + degp[1] + 1.0
    return lax.rsqrt(deg)


def _mm1_body(x_ref, w_ref, b_ref, degp_ref, y_ref, sl_ref):
    xw = jnp.dot(x_ref[...], w_ref[...], preferred_element_type=jnp.float32)
    dinv = _dinv_of(degp_ref[...])
    y_ref[...] = xw * dinv
    sl_ref[...] = xw * (dinv * dinv) + b_ref[...]


@functools.lru_cache(maxsize=None)
def _make_mm1(br, grid, degoff):
    n = br * grid
    return pl.pallas_call(
        _mm1_body,
        grid=(grid,),
        in_specs=[
            pl.BlockSpec((br, D), lambda i: (i, 0)),
            pl.BlockSpec((D, D), lambda i: (0, 0)),
            pl.BlockSpec((1, D), lambda i: (0, 0)),
            pl.BlockSpec((2, br, 1), lambda i: (0, degoff + i, 0)),
        ],
        out_specs=[
            pl.BlockSpec((br, D), lambda i: (i, 0)),
            pl.BlockSpec((br, D), lambda i: (i, 0)),
        ],
        out_shape=[
            jax.ShapeDtypeStruct((n, D), jnp.float32),
            jax.ShapeDtypeStruct((n, D), jnp.float32),
        ],
    )


def _mid_body(aggp_ref, sl_ref, w_ref, b_ref, degp_ref, y_ref, sl2_ref):
    dinv = _dinv_of(degp_ref[...])
    a = aggp_ref[0] + aggp_ref[1]
    h = jnp.maximum(a * dinv + sl_ref[...], 0.0)
    xw = jnp.dot(h, w_ref[...], preferred_element_type=jnp.float32)
    y_ref[...] = xw * dinv
    sl2_ref[...] = xw * (dinv * dinv) + b_ref[...]


@functools.lru_cache(maxsize=None)
def _make_mid(br, grid, off):
    n = br * grid
    return pl.pallas_call(
        _mid_body,
        grid=(grid,),
        in_specs=[
            pl.BlockSpec((2, br, D), lambda i: (0, off + i, 0)),
            pl.BlockSpec((br, D), lambda i: (i, 0)),
            pl.BlockSpec((D, D), lambda i: (0, 0)),
            pl.BlockSpec((1, D), lambda i: (0, 0)),
            pl.BlockSpec((2, br, 1), lambda i: (0, off + i, 0)),
        ],
        out_specs=[
            pl.BlockSpec((br, D), lambda i: (i, 0)),
            pl.BlockSpec((br, D), lambda i: (i, 0)),
        ],
        out_shape=[
            jax.ShapeDtypeStruct((n, D), jnp.float32),
            jax.ShapeDtypeStruct((n, D), jnp.float32),
        ],
    )


def _post_body(aggp_ref, sl_ref, degp_ref, h_ref):
    dinv = _dinv_of(degp_ref[...])
    h_ref[...] = jnp.maximum((aggp_ref[0] + aggp_ref[1]) * dinv + sl_ref[...],
                             0.0)


@functools.lru_cache(maxsize=None)
def _make_post(br, grid, off):
    n = br * grid
    return pl.pallas_call(
        _post_body,
        grid=(grid,),
        in_specs=[
            pl.BlockSpec((2, br, D), lambda i: (0, off + i, 0)),
            pl.BlockSpec((br, D), lambda i: (i, 0)),
            pl.BlockSpec((2, br, 1), lambda i: (0, off + i, 0)),
        ],
        out_specs=pl.BlockSpec((br, D), lambda i: (i, 0)),
        out_shape=jax.ShapeDtypeStruct((n, D), jnp.float32),
    )


def _means_body(s1_ref, c1_ref, s2_ref, c2_ref, xp_ref, x2p_ref, comb_ref):
    c1 = jnp.maximum(c1_ref[0] + c1_ref[1], 1.0)
    c2 = jnp.maximum(c2_ref[0] + c2_ref[1], 1.0)
    xp = (s1_ref[0] + s1_ref[1]) / c1
    x2p = (s2_ref[0] + s2_ref[1]) / c2
    xp_ref[...] = xp
    x2p_ref[...] = x2p
    comb_ref[...] = xp + x2p


_means = pl.pallas_call(
    _means_body,
    in_specs=[
        pl.BlockSpec((2, NSEG, D), lambda: (0, 0, 0)),
        pl.BlockSpec((2, NSEG, 1), lambda: (0, 0, 0)),
        pl.BlockSpec((2, NSEG, D), lambda: (0, 0, 0)),
        pl.BlockSpec((2, NSEG, 1), lambda: (0, 0, 0)),
    ],
    out_specs=[
        pl.BlockSpec((NSEG, D), lambda: (0, 0)),
        pl.BlockSpec((NSEG, D), lambda: (0, 0)),
        pl.BlockSpec((NSEG, D), lambda: (0, 0)),
    ],
    out_shape=[
        jax.ShapeDtypeStruct((NSEG, D), jnp.float32),
        jax.ShapeDtypeStruct((NSEG, D), jnp.float32),
        jax.ShapeDtypeStruct((NSEG, D), jnp.float32),
    ],
)


def _ln(x, g, t):
    m = jnp.mean(x, axis=-1, keepdims=True)
    v = jnp.mean((x - m) ** 2, axis=-1, keepdims=True)
    return (x - m) * lax.rsqrt(v + 1e-5) * g + t


def _attn_body(x_ref, wq, wk, wv, wo, bq, bk, bv, bo, g1, t1, g2, t2,
               wf1, bf1, wf2, bf2, t_ref, aw_ref):
    x = x_ref[0]
    ct = (((1,), (1,)), ((), ()))
    q = lax.dot_general(x, wq[...], ct, preferred_element_type=jnp.float32) + bq[...]
    k = lax.dot_general(x, wk[...], ct, preferred_element_type=jnp.float32) + bk[...]
    v = lax.dot_general(x, wv[...], ct, preferred_element_type=jnp.float32) + bv[...]
    dh = D // H
    scale = 1.0 / jnp.sqrt(jnp.float32(dh))
    o_parts = []
    for h in range(H):
        qh = q[:, h * dh:(h + 1) * dh]
        kh = k[:, h * dh:(h + 1) * dh]
        vh = v[:, h * dh:(h + 1) * dh]
        logits = lax.dot_general(qh, kh, ct, preferred_element_type=jnp.float32) * scale
        m = jnp.max(logits, axis=-1, keepdims=True)
        e = jnp.exp(logits - m)
        aw = e / jnp.sum(e, axis=-1, keepdims=True)
        aw_ref[0, h] = aw
        o_parts.append(jnp.dot(aw, vh, preferred_element_type=jnp.float32))
    o = jnp.concatenate(o_parts, axis=-1)
    o = lax.dot_general(o, wo[...], ct, preferred_element_type=jnp.float32) + bo[...]
    hh = _ln(x + o, g1[...], t1[...])
    ff = jnp.maximum(
        lax.dot_general(hh, wf1[...], ct, preferred_element_type=jnp.float32) + bf1[...],
        0.0)
    ff = lax.dot_general(ff, wf2[...], ct, preferred_element_type=jnp.float32) + bf2[...]
    t_ref[0] = _ln(hh + ff, g2[...], t2[...])


def _make_attn():
    wspec = pl.BlockSpec((D, D), lambda i: (0, 0))
    bspec = pl.BlockSpec((1, D), lambda i: (0, 0))
    return pl.pallas_call(
        _attn_body,
        grid=(B,),
        in_specs=[pl.BlockSpec((1, NR, D), lambda i: (i, 0, 0))]
        + [wspec] * 4 + [bspec] * 4 + [bspec] * 4 + [wspec, bspec, wspec, bspec],
        out_specs=[
            pl.BlockSpec((1, NR, D), lambda i: (i, 0, 0)),
            pl.BlockSpec((1, H, NR, NR), lambda i: (i, 0, 0, 0)),
        ],
        out_shape=[
            jax.ShapeDtypeStruct((B, NR, D), jnp.float32),
            jax.ShapeDtypeStruct((B, H, NR, NR), jnp.float32),
        ],
    )


_attn = _make_attn()

KBLK = 512
KSTEPS = (NR * D) // KBLK


def _mlp_body(x_ref, w1_ref, b1_ref, bng_ref, bnb_ref, w2_ref, b2_ref,
              z_ref, out_ref):
    kk = pl.program_id(0)

    @pl.when(kk == 0)
    def _():
        z_ref[...] = jnp.broadcast_to(b1_ref[...], (B, HID))

    z_ref[...] += jnp.dot(x_ref[...], w1_ref[...], preferred_element_type=jnp.float32)

    @pl.when(kk == KSTEPS - 1)
    def _():
        z = z_ref[...] * (1.0 / jnp.sqrt(1.0 + 1e-5)) * bng_ref[...] + bnb_ref[...]
        z = jnp.where(z > 0, z, 0.01 * z)
        out_ref[...] = jnp.dot(z, w2_ref[...], preferred_element_type=jnp.float32) \
            + b2_ref[...]


_mlp = pl.pallas_call(
    _mlp_body,
    grid=(KSTEPS,),
    in_specs=[
        pl.BlockSpec((B, KBLK), lambda k: (0, k)),
        pl.BlockSpec((KBLK, HID), lambda k: (k, 0)),
        pl.BlockSpec((1, HID), lambda k: (0, 0)),
        pl.BlockSpec((1, HID), lambda k: (0, 0)),
        pl.BlockSpec((1, HID), lambda k: (0, 0)),
        pl.BlockSpec((HID, 128), lambda k: (0, 0)),
        pl.BlockSpec((1, 128), lambda k: (0, 0)),
    ],
    out_specs=[
        pl.BlockSpec((B, HID), lambda k: (0, 0)),
        pl.BlockSpec((B, 128), lambda k: (0, 0)),
    ],
    out_shape=[
        jax.ShapeDtypeStruct((B, HID), jnp.float32),
        jax.ShapeDtypeStruct((B, 128), jnp.float32),
    ],
)


def _pad1(a, n, val=0):
    return jnp.pad(a, (0, n - a.shape[0]), constant_values=val)


def kernel(x, node_roi, edge_index, edge_attr, batch, x2, roi2, edge_index2,
           edge_attr2, batch2, params):
    p = params

    seg1 = batch * NR + node_roi
    seg2 = batch2 * NR + roi2

    scat_idx = jnp.concatenate([
        edge_index[1],
        edge_index2[1] + NP1,
        seg1 + NJ,
        seg2 + (NJ + NPSEG),
    ])
    scat_w = jnp.concatenate([
        edge_attr, edge_attr2,
        jnp.ones((N1,), jnp.float32), jnp.ones((N2,), jnp.float32),
    ])
    scat_idx = _pad1(scat_idx, E_SCAT_PAD)
    scat_w = _pad1(scat_w, E_SCAT_PAD)
    hist = _get_scalar_scatter()(scat_idx, scat_w)
    flat = hist.reshape(NC, HROWS * 128)
    degjp = flat[:, :NJ].reshape(NC, NJ, 1)
    cnt1p = flat[:, NJ:NJ + NSEG].reshape(NC, NSEG, 1)
    cnt2p = flat[:, NJ + NPSEG:NJ + NPSEG + NSEG].reshape(NC, NSEG, 1)

    sj = _pad1(jnp.concatenate([edge_index[0], edge_index2[0] + NP1]), EPADJ)
    dj = _pad1(jnp.concatenate([edge_index[1], edge_index2[1] + NP1]), EPADJ)
    wj = _pad1(jnp.concatenate([edge_attr, edge_attr2]), EPADJ)
    def _rr(a):
        return a.reshape(NCH_AGG, NW, CK).transpose(1, 0, 2)

    ej = jnp.stack([_rr(sj), _rr(dj)], axis=2)
    wjr = _rr(wj)

    segj = jnp.concatenate([
        _pad1(seg1, RP1, NSEG),
        _pad1(seg2, RP2, NSEG) + NPSEG,
    ]).reshape(NW, NCH_POOL, CK)

    _agg = _make_rowagg(NJ, NCH_AGG, True)
    _pool = _make_rowagg(NJSEG, NCH_POOL, False)

    mm1_1 = _make_mm1(632, 16, 0)
    mm1_2 = _make_mm1(CK, 10, 79)
    mid_1 = _make_mid(632, 16, 0)
    mid_2 = _make_mid(CK, 10, 79)
    post_1 = _make_post(632, 16, 0)
    post_2 = _make_post(CK, 10, 79)

    x1p = jnp.pad(x, ((0, NP1 - N1), (0, 0)))
    x2p_in = jnp.pad(x2, ((0, NP2 - N2), (0, 0)))

    b1 = p['b1'].reshape(1, D); b2 = p['b2'].reshape(1, D)
    b1r = p['b1r'].reshape(1, D); b2r = p['b2r'].reshape(1, D)
    y1, sl1 = mm1_1(x1p, p['W1'], b1, degjp)
    y1r, sl1r = mm1_2(x2p_in, p['W1r'], b1r, degjp)
    yj = jnp.concatenate([y1, y1r])
    aggp = _agg(yj, ej, wjr)

    y2, sl2 = mid_1(aggp, sl1, p['W2'], b2, degjp)
    y2r, sl2r = mid_2(aggp, sl1r, p['W2r'], b2r, degjp)
    yj2 = jnp.concatenate([y2, y2r])
    aggp2 = _agg(yj2, ej, wjr)

    h1 = post_1(aggp2, sl2, degjp)
    h2 = post_2(aggp2, sl2r, degjp)
    hcat = jnp.concatenate([
        jnp.pad(h1, ((0, RP1 - NP1), (0, 0))),
        jnp.pad(h2, ((0, RP2 - NP2), (0, 0))),
    ])
    sumsp = _pool(hcat, segj)

    xp2d, x2p2d, comb2d = _means(sumsp[:, :NSEG], cnt1p,
                                 sumsp[:, NPSEG:NPSEG + NSEG], cnt2p)
    xp = xp2d.reshape(B, NR, D)
    x2p = x2p2d.reshape(B, NR, D)
    combined = comb2d.reshape(B, NR, D)

    r = lambda v: v.reshape(1, D)
    t_out, aw = _attn(combined, p['Wq'], p['Wk'], p['Wv'], p['Wo'],
                      r(p['bq']), r(p['bk']), r(p['bv']), r(p['bo']),
                      r(p['g1']), r(p['t1']), r(p['g2']), r(p['t2']),
                      p['Wf1'], r(p['bf1']), p['Wf2'], r(p['bf2']))

    flat_t = t_out.reshape(B, NR * D)
    wc2p = jnp.pad(p['Wc2'], ((0, 0), (0, 128 - OUT)))
    bc2p = jnp.pad(p['bc2'], (0, 128 - OUT)).reshape(1, 128)
    _z, outp = _mlp(flat_t, p['Wc1'], p['bc1'].reshape(1, HID),
                    p['bng'].reshape(1, HID), p['bnb'].reshape(1, HID),
                    wc2p, bc2p)
    out = outp[:, :OUT]

    return (out, xp, x2p, combined, t_out, aw)


# speedup vs baseline: 1.1584x; 1.1584x over previous
import functools

import jax
import jax.numpy as jnp
from jax import lax
from jax.experimental import pallas as pl
from jax.experimental.pallas import tpu as pltpu
from jax.experimental.pallas import tpu_sc as plsc

N1 = 10000; E1 = 320000; N2 = 1184; E2 = 175232
D = 128; H = 4; NR = 148; B = 8; OUT = 2; HID = 1000
NSEG = B * NR

NC, NS = 2, 16
NW = NC * NS
CK = 128

NP1 = 10112
NP2 = 1280
NJ = NP1 + NP2
NPSEG = 1280
NJSEG = 2 * NPSEG

EJ = E1 + E2
NCH_AGG = 124
EPADJ = NW * CK * NCH_AGG
NRING = 4
RP1 = 12288
RP2 = 4096
NCH_POOL = (RP1 + RP2) // (NW * CK)

HROWS = 128
SCK = 2048
E_SCAT = E1 + E2 + N1 + N2
SCHUNKS = -(-E_SCAT // (NW * SCK))
E_SCAT_PAD = NW * SCK * SCHUNKS


@functools.lru_cache(maxsize=None)
def _get_mesh():
    return plsc.VectorSubcoreMesh(core_axis_name="c", subcore_axis_name="s",
                                  num_cores=NC, num_subcores=NS)


def _worker_id():
    return lax.axis_index("c") * NS + lax.axis_index("s")


def _zero_vmem_rows(ref, nrows):
    zeros = jnp.zeros((16,), jnp.float32)

    def body(r, _):
        for g in range(8):
            ref[r, pl.ds(16 * g, 16)] = zeros
        return _

    lax.fori_loop(0, nrows, body, None)


def _zero_shared_slice(shared, zbuf, zrows, row0, nrows):
    off = 0
    while off < nrows:
        sz = min(zrows, nrows - off)
        pltpu.sync_copy(zbuf.at[pl.ds(0, sz)],
                        shared.at[pl.ds(row0 + off, sz)])
        off += sz


def _scalar_scatter_body(idx_hbm, w_hbm, out_hbm, hist, idxb, wb, rowidx, accum):
    c = lax.axis_index("c")
    s = lax.axis_index("s")
    wid = _worker_id()

    _zero_vmem_rows(hist, HROWS)
    pltpu.sync_copy(hist.at[pl.ds(0, 8)], accum.at[pl.ds(s * 8, 8)])
    for g in range(HROWS // 16):
        rowidx[pl.ds(16 * g, 16)] = lax.iota(jnp.int32, 16) + 16 * g
    plsc.subcore_barrier()

    lanes = lax.iota(jnp.int32, 16)

    def chunk(j, _):
        base = (wid * SCHUNKS + j) * SCK
        pltpu.sync_copy(idx_hbm.at[pl.ds(base, SCK)], idxb)
        pltpu.sync_copy(w_hbm.at[pl.ds(base, SCK)], wb)

        def grp(t, _):
            iv = idxb[pl.ds(16 * t, 16)]
            wv = wb[pl.ds(16 * t, 16)]
            for l in range(16):
                i = iv[l]
                r = lax.shift_right_logical(i, 7)
                col = lax.bitwise_and(i, 127)
                colg = lax.bitwise_and(col, 112)
                vec = jnp.where(lanes == col - colg, wv[l], 0.0)
                plsc.addupdate(hist.at[r, pl.ds(colg, 16)], vec)
            return _

        lax.fori_loop(0, SCK // 16, grp, None)
        return _

    lax.fori_loop(0, SCHUNKS, chunk, None)
    pltpu.sync_copy(hist, accum.at[rowidx], add=True)
    plsc.subcore_barrier()
    pltpu.sync_copy(accum.at[pl.ds(s * 8, 8)], out_hbm.at[c, pl.ds(s * 8, 8)])


@functools.lru_cache(maxsize=None)
def _get_scalar_scatter():
    return pl.kernel(
        _scalar_scatter_body,
        out_type=jax.ShapeDtypeStruct((NC, HROWS, 128), jnp.float32),
        mesh=_get_mesh(),
        scratch_types=[
            pltpu.VMEM((HROWS, 128), jnp.float32),
            pltpu.VMEM((SCK,), jnp.int32),
            pltpu.VMEM((SCK,), jnp.float32),
            pltpu.VMEM((HROWS,), jnp.int32),
            pltpu.VMEM_SHARED((HROWS, 128), jnp.float32),
        ],
    )


@functools.lru_cache(maxsize=None)
def _make_rowagg(nrows, nchunks, gather):
    rps = nrows // NS
    assert nchunks % 2 == 0 and rps % 8 == 0

    def body(*refs):
        if gather:
            y_hbm, e_hbm, w_hbm, out_hbm = refs[:4]
            ia, wring, r0, r1, gs0, gs1, accum = refs[4:11]
            isems = refs[11:11 + NRING]
            wsems = refs[11 + NRING:11 + 2 * NRING]
        else:
            y_hbm, d_hbm, out_hbm = refs[:3]
            dall, r0, r1, gs0, gs1, accum = refs[3:]
        c = lax.axis_index("c")
        s = lax.axis_index("s")
        wid = _worker_id()

        if not gather:
            pltpu.sync_copy(d_hbm.at[wid], dall)
        _zero_vmem_rows(r0, CK)
        _zero_shared_slice(accum, r0, CK, s * rps, rps)
        plsc.subcore_barrier()

        def idx_start(j, b):
            pltpu.async_copy(e_hbm.at[wid, j], ia.at[b], isems[b])
            pltpu.async_copy(w_hbm.at[wid, j], wring.at[b], wsems[b])

        def idx_wait(j, b):
            pltpu.make_async_copy(e_hbm.at[wid, j], ia.at[b], isems[b]).wait()
            pltpu.make_async_copy(w_hbm.at[wid, j], wring.at[b],
                                  wsems[b]).wait()

        def g_start(j, b, rbuf, gsem):
            if gather:
                for h in (0, 1):
                    pltpu.async_copy(y_hbm.at[ia.at[b, 0, pl.ds(64 * h, 64)]],
                                     rbuf.at[pl.ds(64 * h, 64)], gsem)
            else:
                base = (wid * nchunks + j) * CK
                pltpu.async_copy(y_hbm.at[pl.ds(base, CK)], rbuf, gsem)

        def g_wait(j, b, rbuf, gsem):
            if gather:
                for h in (0, 1):
                    pltpu.make_async_copy(
                        y_hbm.at[ia.at[b, 0, pl.ds(64 * h, 64)]],
                        rbuf.at[pl.ds(64 * h, 64)], gsem).wait()
            else:
                base = (wid * nchunks + j) * CK
                pltpu.make_async_copy(y_hbm.at[pl.ds(base, CK)], rbuf,
                                      gsem).wait()

        def scale_scatter(j, b, rbuf):
            if gather:
                def scale(t, _):
                    wv = wring[b, pl.ds(16 * t, 16)]
                    for l in range(16):
                        wk = wv[l]
                        rr = 16 * t + l
                        for g in range(8):
                            rbuf[rr, pl.ds(16 * g, 16)] = (
                                rbuf[rr, pl.ds(16 * g, 16)] * wk)
                    return _

                lax.fori_loop(0, CK // 16, scale, None)
                pltpu.sync_copy(rbuf, accum.at[ia.at[b, 1]], add=True)
            else:
                pltpu.sync_copy(rbuf, accum.at[dall.at[j]], add=True)

        if gather:
            assert nchunks % NRING == 0
            rbufs = (r0, r1)
            gsems = (gs0, gs1)
            for b in range(NRING):
                idx_start(b, b)
            idx_wait(0, 0)
            g_start(0, 0, r0, gs0)

            def quad(qq, _):
                j0 = NRING * qq
                for u in range(NRING):
                    j = j0 + u
                    un = (u + 1) % NRING
                    if u + 1 < NRING:
                        idx_wait(j + 1, un)
                        g_start(j + 1, un, rbufs[un % 2], gsems[un % 2])
                    else:
                        @pl.when(j + 1 < nchunks)
                        def _(j=j, un=un):
                            idx_wait(j + 1, un)
                            g_start(j + 1, un, rbufs[un % 2], gsems[un % 2])
                    g_wait(j, u, rbufs[u % 2], gsems[u % 2])
                    scale_scatter(j, u, rbufs[u % 2])

                    @pl.when(j + NRING < nchunks)
                    def _(j=j, u=u):
                        idx_start(j + NRING, u)

                return _

            lax.fori_loop(0, nchunks // NRING, quad, None)
        else:
            g_start(0, 0, r0, gs0)

            def pair(jj, _):
                j0 = 2 * jj
                j1 = j0 + 1
                g_start(j1, 0, r1, gs1)
                g_wait(j0, 0, r0, gs0)
                scale_scatter(j0, 0, r0)

                @pl.when(j0 + 2 < nchunks)
                def _():
                    g_start(j0 + 2, 0, r0, gs0)

                g_wait(j1, 0, r1, gs1)
                scale_scatter(j1, 0, r1)
                return _

            lax.fori_loop(0, nchunks // 2, pair, None)

        plsc.subcore_barrier()
        off = 0
        while off < rps:
            sz = min(CK, rps - off)
            pltpu.sync_copy(accum.at[pl.ds(s * rps + off, sz)],
                            out_hbm.at[c, pl.ds(s * rps + off, sz)])
            off += sz

    scratch = []
    if gather:
        scratch.append(pltpu.VMEM((NRING, 2, CK), jnp.int32))
        scratch.append(pltpu.VMEM((NRING, CK), jnp.float32))
    else:
        scratch.append(pltpu.VMEM((nchunks, CK), jnp.int32))
    scratch += [
        pltpu.VMEM((CK, 128), jnp.float32),
        pltpu.VMEM((CK, 128), jnp.float32),
        pltpu.SemaphoreType.DMA,
        pltpu.SemaphoreType.DMA,
        pltpu.VMEM_SHARED((nrows, 128), jnp.float32),
    ]
    if gather:
        scratch += [pltpu.SemaphoreType.DMA] * (2 * NRING)
    return pl.kernel(
        body,
        out_type=jax.ShapeDtypeStruct((NC, nrows, 128), jnp.float32),
        mesh=_get_mesh(),
        scratch_types=scratch,
    )


def _dinv_of(degp):
    deg = degp[0] + degp[1] + 1.0
    return lax.rsqrt(deg)


def _mm1_body(x_ref, w_ref, b_ref, degp_ref, y_ref, sl_ref):
    xw = jnp.dot(x_ref[...], w_ref[...], preferred_element_type=jnp.float32)
    dinv = _dinv_of(degp_ref[...])
    y_ref[...] = xw * dinv
    sl_ref[...] = xw * (dinv * dinv) + b_ref[...]


@functools.lru_cache(maxsize=None)
def _make_mm1(br, grid, degoff):
    n = br * grid
    return pl.pallas_call(
        _mm1_body,
        grid=(grid,),
        in_specs=[
            pl.BlockSpec((br, D), lambda i: (i, 0)),
            pl.BlockSpec((D, D), lambda i: (0, 0)),
            pl.BlockSpec((1, D), lambda i: (0, 0)),
            pl.BlockSpec((2, br, 1), lambda i: (0, degoff + i, 0)),
        ],
        out_specs=[
            pl.BlockSpec((br, D), lambda i: (i, 0)),
            pl.BlockSpec((br, D), lambda i: (i, 0)),
        ],
        out_shape=[
            jax.ShapeDtypeStruct((n, D), jnp.float32),
            jax.ShapeDtypeStruct((n, D), jnp.float32),
        ],
    )


def _mid_body(aggp_ref, sl_ref, w_ref, b_ref, degp_ref, y_ref, sl2_ref):
    dinv = _dinv_of(degp_ref[...])
    a = aggp_ref[0] + aggp_ref[1]
    h = jnp.maximum(a * dinv + sl_ref[...], 0.0)
    xw = jnp.dot(h, w_ref[...], preferred_element_type=jnp.float32)
    y_ref[...] = xw * dinv
    sl2_ref[...] = xw * (dinv * dinv) + b_ref[...]


@functools.lru_cache(maxsize=None)
def _make_mid(br, grid, off):
    n = br * grid
    return pl.pallas_call(
        _mid_body,
        grid=(grid,),
        in_specs=[
            pl.BlockSpec((2, br, D), lambda i: (0, off + i, 0)),
            pl.BlockSpec((br, D), lambda i: (i, 0)),
            pl.BlockSpec((D, D), lambda i: (0, 0)),
            pl.BlockSpec((1, D), lambda i: (0, 0)),
            pl.BlockSpec((2, br, 1), lambda i: (0, off + i, 0)),
        ],
        out_specs=[
            pl.BlockSpec((br, D), lambda i: (i, 0)),
            pl.BlockSpec((br, D), lambda i: (i, 0)),
        ],
        out_shape=[
            jax.ShapeDtypeStruct((n, D), jnp.float32),
            jax.ShapeDtypeStruct((n, D), jnp.float32),
        ],
    )


def _post_body(aggp_ref, sl_ref, degp_ref, h_ref):
    dinv = _dinv_of(degp_ref[...])
    h_ref[...] = jnp.maximum((aggp_ref[0] + aggp_ref[1]) * dinv + sl_ref[...],
                             0.0)


@functools.lru_cache(maxsize=None)
def _make_post(br, grid, off):
    n = br * grid
    return pl.pallas_call(
        _post_body,
        grid=(grid,),
        in_specs=[
            pl.BlockSpec((2, br, D), lambda i: (0, off + i, 0)),
            pl.BlockSpec((br, D), lambda i: (i, 0)),
            pl.BlockSpec((2, br, 1), lambda i: (0, off + i, 0)),
        ],
        out_specs=pl.BlockSpec((br, D), lambda i: (i, 0)),
        out_shape=jax.ShapeDtypeStruct((n, D), jnp.float32),
    )


def _means_body(s1_ref, c1_ref, s2_ref, c2_ref, xp_ref, x2p_ref, comb_ref):
    c1 = jnp.maximum(c1_ref[0] + c1_ref[1], 1.0)
    c2 = jnp.maximum(c2_ref[0] + c2_ref[1], 1.0)
    xp = (s1_ref[0] + s1_ref[1]) / c1
    x2p = (s2_ref[0] + s2_ref[1]) / c2
    xp_ref[...] = xp
    x2p_ref[...] = x2p
    comb_ref[...] = xp + x2p


_means = pl.pallas_call(
    _means_body,
    in_specs=[
        pl.BlockSpec((2, NSEG, D), lambda: (0, 0, 0)),
        pl.BlockSpec((2, NSEG, 1), lambda: (0, 0, 0)),
        pl.BlockSpec((2, NSEG, D), lambda: (0, 0, 0)),
        pl.BlockSpec((2, NSEG, 1), lambda: (0, 0, 0)),
    ],
    out_specs=[
        pl.BlockSpec((NSEG, D), lambda: (0, 0)),
        pl.BlockSpec((NSEG, D), lambda: (0, 0)),
        pl.BlockSpec((NSEG, D), lambda: (0, 0)),
    ],
    out_shape=[
        jax.ShapeDtypeStruct((NSEG, D), jnp.float32),
        jax.ShapeDtypeStruct((NSEG, D), jnp.float32),
        jax.ShapeDtypeStruct((NSEG, D), jnp.float32),
    ],
)


def _ln(x, g, t):
    m = jnp.mean(x, axis=-1, keepdims=True)
    v = jnp.mean((x - m) ** 2, axis=-1, keepdims=True)
    return (x - m) * lax.rsqrt(v + 1e-5) * g + t


def _attn_body(x_ref, wq, wk, wv, wo, bq, bk, bv, bo, g1, t1, g2, t2,
               wf1, bf1, wf2, bf2, t_ref, aw_ref):
    x = x_ref[0]
    ct = (((1,), (1,)), ((), ()))
    q = lax.dot_general(x, wq[...], ct, preferred_element_type=jnp.float32) + bq[...]
    k = lax.dot_general(x, wk[...], ct, preferred_element_type=jnp.float32) + bk[...]
    v = lax.dot_general(x, wv[...], ct, preferred_element_type=jnp.float32) + bv[...]
    dh = D // H
    scale = 1.0 / jnp.sqrt(jnp.float32(dh))
    o_parts = []
    for h in range(H):
        qh = q[:, h * dh:(h + 1) * dh]
        kh = k[:, h * dh:(h + 1) * dh]
        vh = v[:, h * dh:(h + 1) * dh]
        logits = lax.dot_general(qh, kh, ct, preferred_element_type=jnp.float32) * scale
        m = jnp.max(logits, axis=-1, keepdims=True)
        e = jnp.exp(logits - m)
        aw = e / jnp.sum(e, axis=-1, keepdims=True)
        aw_ref[0, h] = aw
        o_parts.append(jnp.dot(aw, vh, preferred_element_type=jnp.float32))
    o = jnp.concatenate(o_parts, axis=-1)
    o = lax.dot_general(o, wo[...], ct, preferred_element_type=jnp.float32) + bo[...]
    hh = _ln(x + o, g1[...], t1[...])
    ff = jnp.maximum(
        lax.dot_general(hh, wf1[...], ct, preferred_element_type=jnp.float32) + bf1[...],
        0.0)
    ff = lax.dot_general(ff, wf2[...], ct, preferred_element_type=jnp.float32) + bf2[...]
    t_ref[0] = _ln(hh + ff, g2[...], t2[...])


def _make_attn():
    wspec = pl.BlockSpec((D, D), lambda i: (0, 0))
    bspec = pl.BlockSpec((1, D), lambda i: (0, 0))
    return pl.pallas_call(
        _attn_body,
        grid=(B,),
        in_specs=[pl.BlockSpec((1, NR, D), lambda i: (i, 0, 0))]
        + [wspec] * 4 + [bspec] * 4 + [bspec] * 4 + [wspec, bspec, wspec, bspec],
        out_specs=[
            pl.BlockSpec((1, NR, D), lambda i: (i, 0, 0)),
            pl.BlockSpec((1, H, NR, NR), lambda i: (i, 0, 0, 0)),
        ],
        out_shape=[
            jax.ShapeDtypeStruct((B, NR, D), jnp.float32),
            jax.ShapeDtypeStruct((B, H, NR, NR), jnp.float32),
        ],
    )


_attn = _make_attn()

KBLK = 512
KSTEPS = (NR * D) // KBLK


def _mlp_body(x_ref, w1_ref, b1_ref, bng_ref, bnb_ref, w2_ref, b2_ref,
              z_ref, out_ref):
    kk = pl.program_id(0)

    @pl.when(kk == 0)
    def _():
        z_ref[...] = jnp.broadcast_to(b1_ref[...], (B, HID))

    z_ref[...] += jnp.dot(x_ref[...], w1_ref[...], preferred_element_type=jnp.float32)

    @pl.when(kk == KSTEPS - 1)
    def _():
        z = z_ref[...] * (1.0 / jnp.sqrt(1.0 + 1e-5)) * bng_ref[...] + bnb_ref[...]
        z = jnp.where(z > 0, z, 0.01 * z)
        out_ref[...] = jnp.dot(z, w2_ref[...], preferred_element_type=jnp.float32) \
            + b2_ref[...]


_mlp = pl.pallas_call(
    _mlp_body,
    grid=(KSTEPS,),
    in_specs=[
        pl.BlockSpec((B, KBLK), lambda k: (0, k)),
        pl.BlockSpec((KBLK, HID), lambda k: (k, 0)),
        pl.BlockSpec((1, HID), lambda k: (0, 0)),
        pl.BlockSpec((1, HID), lambda k: (0, 0)),
        pl.BlockSpec((1, HID), lambda k: (0, 0)),
        pl.BlockSpec((HID, 128), lambda k: (0, 0)),
        pl.BlockSpec((1, 128), lambda k: (0, 0)),
    ],
    out_specs=[
        pl.BlockSpec((B, HID), lambda k: (0, 0)),
        pl.BlockSpec((B, 128), lambda k: (0, 0)),
    ],
    out_shape=[
        jax.ShapeDtypeStruct((B, HID), jnp.float32),
        jax.ShapeDtypeStruct((B, 128), jnp.float32),
    ],
)


def _pad1(a, n, val=0):
    return jnp.pad(a, (0, n - a.shape[0]), constant_values=val)


def kernel(x, node_roi, edge_index, edge_attr, batch, x2, roi2, edge_index2,
           edge_attr2, batch2, params):
    p = params

    seg1 = batch * NR + node_roi
    seg2 = batch2 * NR + roi2

    scat_idx = jnp.concatenate([
        edge_index[1],
        edge_index2[1] + NP1,
        seg1 + NJ,
        seg2 + (NJ + NPSEG),
    ])
    scat_w = jnp.concatenate([
        edge_attr, edge_attr2,
        jnp.ones((N1,), jnp.float32), jnp.ones((N2,), jnp.float32),
    ])
    scat_idx = _pad1(scat_idx, E_SCAT_PAD)
    scat_w = _pad1(scat_w, E_SCAT_PAD)
    hist = _get_scalar_scatter()(scat_idx, scat_w)
    flat = hist.reshape(NC, HROWS * 128)
    degjp = flat[:, :NJ].reshape(NC, NJ, 1)
    cnt1p = flat[:, NJ:NJ + NSEG].reshape(NC, NSEG, 1)
    cnt2p = flat[:, NJ + NPSEG:NJ + NPSEG + NSEG].reshape(NC, NSEG, 1)

    sj = _pad1(jnp.concatenate([edge_index[0], edge_index2[0] + NP1]), EPADJ)
    dj = _pad1(jnp.concatenate([edge_index[1], edge_index2[1] + NP1]), EPADJ)
    wj = _pad1(jnp.concatenate([edge_attr, edge_attr2]), EPADJ)
    def _rr(a):
        return a.reshape(NCH_AGG, NW, CK).transpose(1, 0, 2)

    ej = jnp.stack([_rr(sj), _rr(dj)], axis=2)
    wjr = _rr(wj)

    segj = jnp.concatenate([
        _pad1(seg1, RP1, NSEG),
        _pad1(seg2, RP2, NSEG) + NPSEG,
    ]).reshape(NW, NCH_POOL, CK)

    _agg = _make_rowagg(NJ, NCH_AGG, True)
    _pool = _make_rowagg(NJSEG, NCH_POOL, False)

    mm1_1 = _make_mm1(632, 16, 0)
    mm1_2 = _make_mm1(CK, 10, 79)
    mid_1 = _make_mid(632, 16, 0)
    mid_2 = _make_mid(CK, 10, 79)
    post_1 = _make_post(632, 16, 0)
    post_2 = _make_post(CK, 10, 79)

    x1p = jnp.pad(x, ((0, NP1 - N1), (0, 0)))
    x2p_in = jnp.pad(x2, ((0, NP2 - N2), (0, 0)))

    b1 = p['b1'].reshape(1, D); b2 = p['b2'].reshape(1, D)
    b1r = p['b1r'].reshape(1, D); b2r = p['b2r'].reshape(1, D)
    y1, sl1 = mm1_1(x1p, p['W1'], b1, degjp)
    y1r, sl1r = mm1_2(x2p_in, p['W1r'], b1r, degjp)
    yj = jnp.concatenate([y1, y1r])
    aggp = _agg(yj, ej, wjr)

    y2, sl2 = mid_1(aggp, sl1, p['W2'], b2, degjp)
    y2r, sl2r = mid_2(aggp, sl1r, p['W2r'], b2r, degjp)
    yj2 = jnp.concatenate([y2, y2r])
    aggp2 = _agg(yj2, ej, wjr)

    h1 = post_1(aggp2, sl2, degjp)
    h2 = post_2(aggp2, sl2r, degjp)
    hcat = jnp.concatenate([
        jnp.pad(h1, ((0, RP1 - NP1), (0, 0))),
        jnp.pad(h2, ((0, RP2 - NP2), (0, 0))),
    ])
    sumsp = _pool(hcat, segj)

    xp2d, x2p2d, comb2d = _means(sumsp[:, :NSEG], cnt1p,
                                 sumsp[:, NPSEG:NPSEG + NSEG], cnt2p)
    xp = xp2d.reshape(B, NR, D)
    x2p = x2p2d.reshape(B, NR, D)
    combined = comb2d.reshape(B, NR, D)

    r = lambda v: v.reshape(1, D)
    t_out, aw = _attn(combined, p['Wq'], p['Wk'], p['Wv'], p['Wo'],
                      r(p['bq']), r(p['bk']), r(p['bv']), r(p['bo']),
                      r(p['g1']), r(p['t1']), r(p['g2']), r(p['t2']),
                      p['Wf1'], r(p['bf1']), p['Wf2'], r(p['bf2']))

    flat_t = t_out.reshape(B, NR * D)
    wc2p = jnp.pad(p['Wc2'], ((0, 0), (0, 128 - OUT)))
    bc2p = jnp.pad(p['bc2'], (0, 128 - OUT)).reshape(1, 128)
    _z, outp = _mlp(flat_t, p['Wc1'], p['bc1'].reshape(1, HID),
                    p['bng'].reshape(1, HID), p['bnb'].reshape(1, HID),
                    wc2p, bc2p)
    out = outp[:, :OUT]

    return (out, xp, x2p, combined, t_out, aw)


# speedup vs baseline: 1.1659x; 1.0064x over previous
import functools

import jax
import jax.numpy as jnp
from jax import lax
from jax.experimental import pallas as pl
from jax.experimental.pallas import tpu as pltpu
from jax.experimental.pallas import tpu_sc as plsc

N1 = 10000; E1 = 320000; N2 = 1184; E2 = 175232
D = 128; H = 4; NR = 148; B = 8; OUT = 2; HID = 1000
NSEG = B * NR

NC, NS = 2, 16
NW = NC * NS
CK = 128

NP1 = 10112
NP2 = 1280
NJ = NP1 + NP2
NPSEG = 1280
NJSEG = 2 * NPSEG

EJ = E1 + E2
NCH_AGG = 124
EPADJ = NW * CK * NCH_AGG
NRING = 4
RP1 = 12288
RP2 = 4096
NCH_POOL = (RP1 + RP2) // (NW * CK)

HROWS = 128
SCK = 2048
E_SCAT = E1 + E2 + N1 + N2
SCHUNKS = -(-E_SCAT // (NW * SCK))
E_SCAT_PAD = NW * SCK * SCHUNKS


@functools.lru_cache(maxsize=None)
def _get_mesh():
    return plsc.VectorSubcoreMesh(core_axis_name="c", subcore_axis_name="s",
                                  num_cores=NC, num_subcores=NS)


def _worker_id():
    return lax.axis_index("c") * NS + lax.axis_index("s")


def _zero_vmem_rows(ref, nrows):
    zeros = jnp.zeros((16,), jnp.float32)

    def body(r, _):
        for g in range(8):
            ref[r, pl.ds(16 * g, 16)] = zeros
        return _

    lax.fori_loop(0, nrows, body, None)


def _zero_shared_slice(shared, zbuf, zrows, row0, nrows):
    off = 0
    while off < nrows:
        sz = min(zrows, nrows - off)
        pltpu.sync_copy(zbuf.at[pl.ds(0, sz)],
                        shared.at[pl.ds(row0 + off, sz)])
        off += sz


def _scalar_scatter_body(idx_hbm, w_hbm, out_hbm, hist, idxb, wb, rowidx, accum):
    c = lax.axis_index("c")
    s = lax.axis_index("s")
    wid = _worker_id()

    _zero_vmem_rows(hist, HROWS)
    pltpu.sync_copy(hist.at[pl.ds(0, 8)], accum.at[pl.ds(s * 8, 8)])
    for g in range(HROWS // 16):
        rowidx[pl.ds(16 * g, 16)] = lax.iota(jnp.int32, 16) + 16 * g
    plsc.subcore_barrier()

    lanes = lax.iota(jnp.int32, 16)

    def chunk(j, _):
        base = (wid * SCHUNKS + j) * SCK
        pltpu.sync_copy(idx_hbm.at[pl.ds(base, SCK)], idxb)
        pltpu.sync_copy(w_hbm.at[pl.ds(base, SCK)], wb)

        def grp(t, _):
            iv = idxb[pl.ds(16 * t, 16)]
            wv = wb[pl.ds(16 * t, 16)]
            for l in range(16):
                i = iv[l]
                r = lax.shift_right_logical(i, 7)
                col = lax.bitwise_and(i, 127)
                colg = lax.bitwise_and(col, 112)
                vec = jnp.where(lanes == col - colg, wv[l], 0.0)
                plsc.addupdate(hist.at[r, pl.ds(colg, 16)], vec)
            return _

        lax.fori_loop(0, SCK // 16, grp, None)
        return _

    lax.fori_loop(0, SCHUNKS, chunk, None)
    pltpu.sync_copy(hist, accum.at[rowidx], add=True)
    plsc.subcore_barrier()
    pltpu.sync_copy(accum.at[pl.ds(s * 8, 8)], out_hbm.at[c, pl.ds(s * 8, 8)])


@functools.lru_cache(maxsize=None)
def _get_scalar_scatter():
    return pl.kernel(
        _scalar_scatter_body,
        out_type=jax.ShapeDtypeStruct((NC, HROWS, 128), jnp.float32),
        mesh=_get_mesh(),
        scratch_types=[
            pltpu.VMEM((HROWS, 128), jnp.float32),
            pltpu.VMEM((SCK,), jnp.int32),
            pltpu.VMEM((SCK,), jnp.float32),
            pltpu.VMEM((HROWS,), jnp.int32),
            pltpu.VMEM_SHARED((HROWS, 128), jnp.float32),
        ],
    )


@functools.lru_cache(maxsize=None)
def _make_rowagg(nrows, nchunks, gather):
    rps = nrows // NS
    assert nchunks % 2 == 0 and rps % 8 == 0

    def body(*refs):
        if gather:
            y_hbm, e_hbm, w_hbm, out_hbm = refs[:4]
            ia, wring, r0, r1, gs0, gs1, accum = refs[4:11]
            isems = refs[11:11 + NRING]
            wsems = refs[11 + NRING:11 + 2 * NRING]
        else:
            y_hbm, d_hbm, out_hbm = refs[:3]
            dall, r0, r1, gs0, gs1, accum = refs[3:]
        c = lax.axis_index("c")
        s = lax.axis_index("s")
        wid = _worker_id()

        if not gather:
            pltpu.sync_copy(d_hbm.at[wid], dall)
        _zero_vmem_rows(r0, CK)
        _zero_shared_slice(accum, r0, CK, s * rps, rps)
        plsc.subcore_barrier()

        def idx_start(j, b):
            pltpu.async_copy(e_hbm.at[wid, j], ia.at[b], isems[b])
            pltpu.async_copy(w_hbm.at[wid, j], wring.at[b], wsems[b])

        def idx_wait(j, b):
            pltpu.make_async_copy(e_hbm.at[wid, j], ia.at[b], isems[b]).wait()
            pltpu.make_async_copy(w_hbm.at[wid, j], wring.at[b],
                                  wsems[b]).wait()

        def g_start(j, b, rbuf, gsem):
            if gather:
                pltpu.async_copy(y_hbm.at[ia.at[b, 0]], rbuf, gsem)
            else:
                base = (wid * nchunks + j) * CK
                pltpu.async_copy(y_hbm.at[pl.ds(base, CK)], rbuf, gsem)

        def g_wait(j, b, rbuf, gsem):
            if gather:
                pltpu.make_async_copy(y_hbm.at[ia.at[b, 0]], rbuf, gsem).wait()
            else:
                base = (wid * nchunks + j) * CK
                pltpu.make_async_copy(y_hbm.at[pl.ds(base, CK)], rbuf,
                                      gsem).wait()

        def scale_scatter(j, b, rbuf):
            if gather:
                def scale(t, _):
                    wv = wring[b, pl.ds(16 * t, 16)]
                    for l in range(16):
                        wk = wv[l]
                        rr = 16 * t + l
                        for g in range(8):
                            rbuf[rr, pl.ds(16 * g, 16)] = (
                                rbuf[rr, pl.ds(16 * g, 16)] * wk)
                    return _

                lax.fori_loop(0, CK // 16, scale, None)
                pltpu.sync_copy(rbuf, accum.at[ia.at[b, 1]], add=True)
            else:
                pltpu.sync_copy(rbuf, accum.at[dall.at[j]], add=True)

        if gather:
            assert nchunks % NRING == 0
            rbufs = (r0, r1)
            gsems = (gs0, gs1)
            for b in range(NRING):
                idx_start(b, b)
            idx_wait(0, 0)
            g_start(0, 0, r0, gs0)

            def quad(qq, _):
                j0 = NRING * qq
                for u in range(NRING):
                    j = j0 + u
                    un = (u + 1) % NRING
                    if u + 1 < NRING:
                        idx_wait(j + 1, un)
                        g_start(j + 1, un, rbufs[un % 2], gsems[un % 2])
                    else:
                        @pl.when(j + 1 < nchunks)
                        def _(j=j, un=un):
                            idx_wait(j + 1, un)
                            g_start(j + 1, un, rbufs[un % 2], gsems[un % 2])
                    g_wait(j, u, rbufs[u % 2], gsems[u % 2])
                    scale_scatter(j, u, rbufs[u % 2])

                    @pl.when(j + NRING < nchunks)
                    def _(j=j, u=u):
                        idx_start(j + NRING, u)

                return _

            lax.fori_loop(0, nchunks // NRING, quad, None)
        else:
            g_start(0, 0, r0, gs0)

            def pair(jj, _):
                j0 = 2 * jj
                j1 = j0 + 1
                g_start(j1, 0, r1, gs1)
                g_wait(j0, 0, r0, gs0)
                scale_scatter(j0, 0, r0)

                @pl.when(j0 + 2 < nchunks)
                def _():
                    g_start(j0 + 2, 0, r0, gs0)

                g_wait(j1, 0, r1, gs1)
                scale_scatter(j1, 0, r1)
                return _

            lax.fori_loop(0, nchunks // 2, pair, None)

        plsc.subcore_barrier()
        off = 0
        while off < rps:
            sz = min(CK, rps - off)
            pltpu.sync_copy(accum.at[pl.ds(s * rps + off, sz)],
                            out_hbm.at[c, pl.ds(s * rps + off, sz)])
            off += sz

    scratch = []
    if gather:
        scratch.append(pltpu.VMEM((NRING, 2, CK), jnp.int32))
        scratch.append(pltpu.VMEM((NRING, CK), jnp.float32))
    else:
        scratch.append(pltpu.VMEM((nchunks, CK), jnp.int32))
    scratch += [
        pltpu.VMEM((CK, 128), jnp.float32),
        pltpu.VMEM((CK, 128), jnp.float32),
        pltpu.SemaphoreType.DMA,
        pltpu.SemaphoreType.DMA,
        pltpu.VMEM_SHARED((nrows, 128), jnp.float32),
    ]
    if gather:
        scratch += [pltpu.SemaphoreType.DMA] * (2 * NRING)
    return pl.kernel(
        body,
        out_type=jax.ShapeDtypeStruct((NC, nrows, 128), jnp.float32),
        mesh=_get_mesh(),
        scratch_types=scratch,
    )


NCHP = 3
CH1 = NP1 // CK
CH2 = CH1 + NP2 // CK


@functools.lru_cache(maxsize=None)
def _make_pool2():
    rps = NJSEG // NS

    def body(h1_hbm, h2_hbm, d_hbm, out_hbm, dall, r0, accum):
        c = lax.axis_index("c")
        s = lax.axis_index("s")
        wid = _worker_id()

        pltpu.sync_copy(d_hbm.at[wid], dall)
        _zero_vmem_rows(r0, CK)
        _zero_shared_slice(accum, r0, CK, s * rps, rps)
        plsc.subcore_barrier()

        for j in range(NCHP):
            g = wid * NCHP + j

            @pl.when(g < CH1)
            def _(j=j, g=g):
                pltpu.sync_copy(h1_hbm.at[pl.ds(g * CK, CK)], r0)
                pltpu.sync_copy(r0, accum.at[dall.at[j]], add=True)

            @pl.when(jnp.logical_and(g >= CH1, g < CH2))
            def _(j=j, g=g):
                pltpu.sync_copy(h2_hbm.at[pl.ds((g - CH1) * CK, CK)], r0)
                pltpu.sync_copy(r0, accum.at[dall.at[j]], add=True)

        plsc.subcore_barrier()
        off = 0
        while off < rps:
            sz = min(CK, rps - off)
            pltpu.sync_copy(accum.at[pl.ds(s * rps + off, sz)],
                            out_hbm.at[c, pl.ds(s * rps + off, sz)])
            off += sz

    return pl.kernel(
        body,
        out_type=jax.ShapeDtypeStruct((NC, NJSEG, 128), jnp.float32),
        mesh=_get_mesh(),
        scratch_types=[
            pltpu.VMEM((NCHP, CK), jnp.int32),
            pltpu.VMEM((CK, 128), jnp.float32),
            pltpu.VMEM_SHARED((NJSEG, 128), jnp.float32),
        ],
    )


def _dinv_of(degp):
    deg = degp[0] + degp[1] + 1.0
    return lax.rsqrt(deg)


def _mm1_body(x_ref, w_ref, b_ref, degp_ref, y_ref, sl_ref):
    xw = jnp.dot(x_ref[...], w_ref[...], preferred_element_type=jnp.float32)
    dinv = _dinv_of(degp_ref[...])
    y_ref[...] = xw * dinv
    sl_ref[...] = xw * (dinv * dinv) + b_ref[...]


@functools.lru_cache(maxsize=None)
def _make_mm1(br, grid, degoff):
    n = br * grid
    return pl.pallas_call(
        _mm1_body,
        grid=(grid,),
        in_specs=[
            pl.BlockSpec((br, D), lambda i: (i, 0)),
            pl.BlockSpec((D, D), lambda i: (0, 0)),
            pl.BlockSpec((1, D), lambda i: (0, 0)),
            pl.BlockSpec((2, br, 1), lambda i: (0, degoff + i, 0)),
        ],
        out_specs=[
            pl.BlockSpec((br, D), lambda i: (i, 0)),
            pl.BlockSpec((br, D), lambda i: (i, 0)),
        ],
        out_shape=[
            jax.ShapeDtypeStruct((n, D), jnp.float32),
            jax.ShapeDtypeStruct((n, D), jnp.float32),
        ],
    )


def _mid_body(aggp_ref, sl_ref, w_ref, b_ref, degp_ref, y_ref, sl2_ref):
    dinv = _dinv_of(degp_ref[...])
    a = aggp_ref[0] + aggp_ref[1]
    h = jnp.maximum(a * dinv + sl_ref[...], 0.0)
    xw = jnp.dot(h, w_ref[...], preferred_element_type=jnp.float32)
    y_ref[...] = xw * dinv
    sl2_ref[...] = xw * (dinv * dinv) + b_ref[...]


@functools.lru_cache(maxsize=None)
def _make_mid(br, grid, off):
    n = br * grid
    return pl.pallas_call(
        _mid_body,
        grid=(grid,),
        in_specs=[
            pl.BlockSpec((2, br, D), lambda i: (0, off + i, 0)),
            pl.BlockSpec((br, D), lambda i: (i, 0)),
            pl.BlockSpec((D, D), lambda i: (0, 0)),
            pl.BlockSpec((1, D), lambda i: (0, 0)),
            pl.BlockSpec((2, br, 1), lambda i: (0, off + i, 0)),
        ],
        out_specs=[
            pl.BlockSpec((br, D), lambda i: (i, 0)),
            pl.BlockSpec((br, D), lambda i: (i, 0)),
        ],
        out_shape=[
            jax.ShapeDtypeStruct((n, D), jnp.float32),
            jax.ShapeDtypeStruct((n, D), jnp.float32),
        ],
    )


def _post_body(aggp_ref, sl_ref, degp_ref, h_ref):
    dinv = _dinv_of(degp_ref[...])
    h_ref[...] = jnp.maximum((aggp_ref[0] + aggp_ref[1]) * dinv + sl_ref[...],
                             0.0)


@functools.lru_cache(maxsize=None)
def _make_post(br, grid, off):
    n = br * grid
    return pl.pallas_call(
        _post_body,
        grid=(grid,),
        in_specs=[
            pl.BlockSpec((2, br, D), lambda i: (0, off + i, 0)),
            pl.BlockSpec((br, D), lambda i: (i, 0)),
            pl.BlockSpec((2, br, 1), lambda i: (0, off + i, 0)),
        ],
        out_specs=pl.BlockSpec((br, D), lambda i: (i, 0)),
        out_shape=jax.ShapeDtypeStruct((n, D), jnp.float32),
    )


def _means_body(s1_ref, c1_ref, s2_ref, c2_ref, xp_ref, x2p_ref, comb_ref):
    c1 = jnp.maximum(c1_ref[0] + c1_ref[1], 1.0)
    c2 = jnp.maximum(c2_ref[0] + c2_ref[1], 1.0)
    xp = (s1_ref[0] + s1_ref[1]) / c1
    x2p = (s2_ref[0] + s2_ref[1]) / c2
    xp_ref[...] = xp
    x2p_ref[...] = x2p
    comb_ref[...] = xp + x2p


_means = pl.pallas_call(
    _means_body,
    in_specs=[
        pl.BlockSpec((2, NSEG, D), lambda: (0, 0, 0)),
        pl.BlockSpec((2, NSEG, 1), lambda: (0, 0, 0)),
        pl.BlockSpec((2, NSEG, D), lambda: (0, 0, 0)),
        pl.BlockSpec((2, NSEG, 1), lambda: (0, 0, 0)),
    ],
    out_specs=[
        pl.BlockSpec((NSEG, D), lambda: (0, 0)),
        pl.BlockSpec((NSEG, D), lambda: (0, 0)),
        pl.BlockSpec((NSEG, D), lambda: (0, 0)),
    ],
    out_shape=[
        jax.ShapeDtypeStruct((NSEG, D), jnp.float32),
        jax.ShapeDtypeStruct((NSEG, D), jnp.float32),
        jax.ShapeDtypeStruct((NSEG, D), jnp.float32),
    ],
)


def _ln(x, g, t):
    m = jnp.mean(x, axis=-1, keepdims=True)
    v = jnp.mean((x - m) ** 2, axis=-1, keepdims=True)
    return (x - m) * lax.rsqrt(v + 1e-5) * g + t


def _attn_body(x_ref, wq, wk, wv, wo, bq, bk, bv, bo, g1, t1, g2, t2,
               wf1, bf1, wf2, bf2, t_ref, aw_ref):
    x = x_ref[0]
    ct = (((1,), (1,)), ((), ()))
    q = lax.dot_general(x, wq[...], ct, preferred_element_type=jnp.float32) + bq[...]
    k = lax.dot_general(x, wk[...], ct, preferred_element_type=jnp.float32) + bk[...]
    v = lax.dot_general(x, wv[...], ct, preferred_element_type=jnp.float32) + bv[...]
    dh = D // H
    scale = 1.0 / jnp.sqrt(jnp.float32(dh))
    o_parts = []
    for h in range(H):
        qh = q[:, h * dh:(h + 1) * dh]
        kh = k[:, h * dh:(h + 1) * dh]
        vh = v[:, h * dh:(h + 1) * dh]
        logits = lax.dot_general(qh, kh, ct, preferred_element_type=jnp.float32) * scale
        m = jnp.max(logits, axis=-1, keepdims=True)
        e = jnp.exp(logits - m)
        aw = e / jnp.sum(e, axis=-1, keepdims=True)
        aw_ref[0, h] = aw
        o_parts.append(jnp.dot(aw, vh, preferred_element_type=jnp.float32))
    o = jnp.concatenate(o_parts, axis=-1)
    o = lax.dot_general(o, wo[...], ct, preferred_element_type=jnp.float32) + bo[...]
    hh = _ln(x + o, g1[...], t1[...])
    ff = jnp.maximum(
        lax.dot_general(hh, wf1[...], ct, preferred_element_type=jnp.float32) + bf1[...],
        0.0)
    ff = lax.dot_general(ff, wf2[...], ct, preferred_element_type=jnp.float32) + bf2[...]
    t_ref[0] = _ln(hh + ff, g2[...], t2[...])


def _make_attn():
    wspec = pl.BlockSpec((D, D), lambda i: (0, 0))
    bspec = pl.BlockSpec((1, D), lambda i: (0, 0))
    return pl.pallas_call(
        _attn_body,
        grid=(B,),
        in_specs=[pl.BlockSpec((1, NR, D), lambda i: (i, 0, 0))]
        + [wspec] * 4 + [bspec] * 4 + [bspec] * 4 + [wspec, bspec, wspec, bspec],
        out_specs=[
            pl.BlockSpec((1, NR, D), lambda i: (i, 0, 0)),
            pl.BlockSpec((1, H, NR, NR), lambda i: (i, 0, 0, 0)),
        ],
        out_shape=[
            jax.ShapeDtypeStruct((B, NR, D), jnp.float32),
            jax.ShapeDtypeStruct((B, H, NR, NR), jnp.float32),
        ],
    )


_attn = _make_attn()

KBLK = 512
KSTEPS = (NR * D) // KBLK


def _mlp_body(x_ref, w1_ref, b1_ref, bng_ref, bnb_ref, w2_ref, b2_ref,
              z_ref, out_ref):
    kk = pl.program_id(0)

    @pl.when(kk == 0)
    def _():
        z_ref[...] = jnp.broadcast_to(b1_ref[...], (B, HID))

    z_ref[...] += jnp.dot(x_ref[...], w1_ref[...], preferred_element_type=jnp.float32)

    @pl.when(kk == KSTEPS - 1)
    def _():
        z = z_ref[...] * (1.0 / jnp.sqrt(1.0 + 1e-5)) * bng_ref[...] + bnb_ref[...]
        z = jnp.where(z > 0, z, 0.01 * z)
        out_ref[...] = jnp.dot(z, w2_ref[...], preferred_element_type=jnp.float32) \
            + b2_ref[...]


_mlp = pl.pallas_call(
    _mlp_body,
    grid=(KSTEPS,),
    in_specs=[
        pl.BlockSpec((B, KBLK), lambda k: (0, k)),
        pl.BlockSpec((KBLK, HID), lambda k: (k, 0)),
        pl.BlockSpec((1, HID), lambda k: (0, 0)),
        pl.BlockSpec((1, HID), lambda k: (0, 0)),
        pl.BlockSpec((1, HID), lambda k: (0, 0)),
        pl.BlockSpec((HID, 128), lambda k: (0, 0)),
        pl.BlockSpec((1, 128), lambda k: (0, 0)),
    ],
    out_specs=[
        pl.BlockSpec((B, HID), lambda k: (0, 0)),
        pl.BlockSpec((B, 128), lambda k: (0, 0)),
    ],
    out_shape=[
        jax.ShapeDtypeStruct((B, HID), jnp.float32),
        jax.ShapeDtypeStruct((B, 128), jnp.float32),
    ],
)


def _pad1(a, n, val=0):
    return jnp.pad(a, (0, n - a.shape[0]), constant_values=val)


def kernel(x, node_roi, edge_index, edge_attr, batch, x2, roi2, edge_index2,
           edge_attr2, batch2, params):
    p = params

    seg1 = batch * NR + node_roi
    seg2 = batch2 * NR + roi2

    scat_idx = jnp.concatenate([
        edge_index[1],
        edge_index2[1] + NP1,
        seg1 + NJ,
        seg2 + (NJ + NPSEG),
    ])
    scat_w = jnp.concatenate([
        edge_attr, edge_attr2,
        jnp.ones((N1,), jnp.float32), jnp.ones((N2,), jnp.float32),
    ])
    scat_idx = _pad1(scat_idx, E_SCAT_PAD)
    scat_w = _pad1(scat_w, E_SCAT_PAD)
    hist = _get_scalar_scatter()(scat_idx, scat_w)
    flat = hist.reshape(NC, HROWS * 128)
    degjp = flat[:, :NJ].reshape(NC, NJ, 1)
    cnt1p = flat[:, NJ:NJ + NSEG].reshape(NC, NSEG, 1)
    cnt2p = flat[:, NJ + NPSEG:NJ + NPSEG + NSEG].reshape(NC, NSEG, 1)

    sj = _pad1(jnp.concatenate([edge_index[0], edge_index2[0] + NP1]), EPADJ)
    dj = _pad1(jnp.concatenate([edge_index[1], edge_index2[1] + NP1]), EPADJ)
    wj = _pad1(jnp.concatenate([edge_attr, edge_attr2]), EPADJ)
    def _rr(a):
        return a.reshape(NCH_AGG, NW, CK).transpose(1, 0, 2)

    ej = jnp.stack([_rr(sj), _rr(dj)], axis=2)
    wjr = _rr(wj)

    segj = jnp.concatenate([
        _pad1(seg1, NP1, NSEG),
        _pad1(seg2, NP2, NSEG) + NPSEG,
        jnp.full((NW * NCHP * CK - NJ,), NSEG, jnp.int32),
    ]).reshape(NW, NCHP, CK)

    _agg = _make_rowagg(NJ, NCH_AGG, True)
    _pool = _make_pool2()

    mm1_1 = _make_mm1(632, 16, 0)
    mm1_2 = _make_mm1(CK, 10, 79)
    mid_1 = _make_mid(632, 16, 0)
    mid_2 = _make_mid(CK, 10, 79)
    post_1 = _make_post(632, 16, 0)
    post_2 = _make_post(CK, 10, 79)

    x1p = jnp.pad(x, ((0, NP1 - N1), (0, 0)))
    x2p_in = jnp.pad(x2, ((0, NP2 - N2), (0, 0)))

    b1 = p['b1'].reshape(1, D); b2 = p['b2'].reshape(1, D)
    b1r = p['b1r'].reshape(1, D); b2r = p['b2r'].reshape(1, D)
    y1, sl1 = mm1_1(x1p, p['W1'], b1, degjp)
    y1r, sl1r = mm1_2(x2p_in, p['W1r'], b1r, degjp)
    yj = jnp.concatenate([y1, y1r])
    aggp = _agg(yj, ej, wjr)

    y2, sl2 = mid_1(aggp, sl1, p['W2'], b2, degjp)
    y2r, sl2r = mid_2(aggp, sl1r, p['W2r'], b2r, degjp)
    yj2 = jnp.concatenate([y2, y2r])
    aggp2 = _agg(yj2, ej, wjr)

    h1 = post_1(aggp2, sl2, degjp)
    h2 = post_2(aggp2, sl2r, degjp)
    sumsp = _pool(h1, h2, segj)

    xp2d, x2p2d, comb2d = _means(sumsp[:, :NSEG], cnt1p,
                                 sumsp[:, NPSEG:NPSEG + NSEG], cnt2p)
    xp = xp2d.reshape(B, NR, D)
    x2p = x2p2d.reshape(B, NR, D)
    combined = comb2d.reshape(B, NR, D)

    r = lambda v: v.reshape(1, D)
    t_out, aw = _attn(combined, p['Wq'], p['Wk'], p['Wv'], p['Wo'],
                      r(p['bq']), r(p['bk']), r(p['bv']), r(p['bo']),
                      r(p['g1']), r(p['t1']), r(p['g2']), r(p['t2']),
                      p['Wf1'], r(p['bf1']), p['Wf2'], r(p['bf2']))

    flat_t = t_out.reshape(B, NR * D)
    wc2p = jnp.pad(p['Wc2'], ((0, 0), (0, 128 - OUT)))
    bc2p = jnp.pad(p['bc2'], (0, 128 - OUT)).reshape(1, 128)
    _z, outp = _mlp(flat_t, p['Wc1'], p['bc1'].reshape(1, HID),
                    p['bng'].reshape(1, HID), p['bnb'].reshape(1, HID),
                    wc2p, bc2p)
    out = outp[:, :OUT]

    return (out, xp, x2p, combined, t_out, aw)
